# baseline (speedup 1.0000x reference)
import functools

import jax
import jax.numpy as jnp
from jax import lax
from jax.experimental import pallas as pl
from jax.experimental.pallas import tpu as pltpu
from jax.experimental.pallas import tpu_sc as plsc

B = 4096
L = 200
N_TOK = B * L
EDIM = 16
TOK_DIM = 64
VOCAB = 1000000
GROWS = B * EDIM // 128

NUM_WORKERS = 32
PER_W = N_TOK // NUM_WORKERS
TBATCH = 128
K_INFLIGHT = 20
GROUP = TBATCH * K_INFLIGHT
NUM_GROUPS = PER_W // GROUP


@functools.cache
def _make_sc_gather():
    mesh = plsc.VectorSubcoreMesh(core_axis_name="c", subcore_axis_name="s")

    @functools.partial(
        pl.kernel,
        mesh=mesh,
        out_type=[
            jax.ShapeDtypeStruct((N_TOK, EDIM), jnp.float32),
            jax.ShapeDtypeStruct((N_TOK, EDIM), jnp.float32),
        ],
        scratch_types=[
            pltpu.VMEM((GROUP,), jnp.int32),
            pltpu.VMEM((GROUP,), jnp.int32),
            pltpu.VMEM((GROUP, EDIM), jnp.float32),
            pltpu.VMEM((GROUP, EDIM), jnp.float32),
            pltpu.SemaphoreType.DMA,
            pltpu.SemaphoreType.DMA,
        ],
        compiler_params=pltpu.CompilerParams(use_tc_tiling_on_sc=False),
    )
    def _sc_gather(ridx_hbm, aidx_hbm, rtab_hbm, atab_hbm, out_r_hbm,
                   out_a_hbm, idxr_v, idxa_v, bufr_v, bufa_v, semr, sema):
        wid = lax.axis_index("s") * 2 + lax.axis_index("c")
        base = wid * PER_W

        def group_body(g, carry):
            goff = base + g * GROUP
            pltpu.sync_copy(ridx_hbm.at[pl.ds(goff, GROUP)], idxr_v)
            pltpu.sync_copy(aidx_hbm.at[pl.ds(goff, GROUP)], idxa_v)
            rcopies = []
            acopies = []
            for t in range(K_INFLIGHT):
                sl = pl.ds(t * TBATCH, TBATCH)
                rcopies.append(
                    pltpu.async_copy(rtab_hbm.at[idxr_v.at[sl]], bufr_v.at[sl],
                                     semr))
                acopies.append(
                    pltpu.async_copy(atab_hbm.at[idxa_v.at[sl]], bufa_v.at[sl],
                                     sema))
            for c in rcopies:
                c.wait()
            pltpu.sync_copy(bufr_v, out_r_hbm.at[pl.ds(goff, GROUP)])
            for c in acopies:
                c.wait()
            pltpu.sync_copy(bufa_v, out_a_hbm.at[pl.ds(goff, GROUP)])
            return carry

        lax.fori_loop(0, NUM_GROUPS, group_body, 0)

    return _sc_gather


def _proj_body(gr_ref, ga_ref, ct_ref, wr_ref, wa_ref, wc_ref, bb_ref,
               o_ref):
    acc = jnp.dot(gr_ref[...], wr_ref[...], preferred_element_type=jnp.float32)
    acc = acc + jnp.dot(ga_ref[...], wa_ref[...],
                        preferred_element_type=jnp.float32)
    acc = acc + jnp.dot(ct_ref[...], wc_ref[...],
                        preferred_element_type=jnp.float32)
    o_ref[...] = acc + bb_ref[...]


def _tc_project(gr128, ga128, ct512, wbd_r, wbd_a, wbd_c, bb):
    return pl.pallas_call(
        _proj_body,
        grid=(L,),
        in_specs=[
            pl.BlockSpec((GROWS, 128), lambda i: (i, 0)),
            pl.BlockSpec((GROWS, 128), lambda i: (i, 0)),
            pl.BlockSpec((GROWS, EDIM), lambda i: (i, 0)),
            pl.BlockSpec((128, 8 * TOK_DIM), lambda i: (0, 0)),
            pl.BlockSpec((128, 8 * TOK_DIM), lambda i: (0, 0)),
            pl.BlockSpec((EDIM, 8 * TOK_DIM), lambda i: (0, 0)),
            pl.BlockSpec((1, 8 * TOK_DIM), lambda i: (0, 0)),
        ],
        out_specs=pl.BlockSpec((GROWS, 8 * TOK_DIM), lambda i: (i, 0)),
        out_shape=jax.ShapeDtypeStruct((L * GROWS, 8 * TOK_DIM), jnp.float32),
        compiler_params=pltpu.CompilerParams(
            dimension_semantics=("arbitrary",),
        ),
    )(gr128, ga128, ct512, wbd_r, wbd_a, wbd_c, bb)


def kernel(x, resp_table, act_table, W, b):
    ridx = x[:, :, 0].T.reshape(N_TOK).astype(jnp.int32)
    aidx = x[:, :, 1].T.reshape(N_TOK).astype(jnp.int32)
    ct512 = jnp.transpose(x[:, :, 2:4], (1, 0, 2)).reshape(L * GROWS, EDIM)
    gr, ga = _make_sc_gather()(ridx, aidx, resp_table, act_table)
    gr128 = gr.reshape(L * GROWS, 128)
    ga128 = ga.reshape(L * GROWS, 128)
    eye8 = jnp.eye(8, dtype=jnp.float32)
    wbd_r = jnp.kron(eye8, W[0:16])
    wbd_a = jnp.kron(eye8, W[16:32])
    wbd_c = jnp.kron(eye8, W[32:34])
    bb = jnp.tile(b, 8).reshape(1, 8 * TOK_DIM)
    out128 = _tc_project(gr128, ga128, ct512, wbd_r, wbd_a, wbd_c, bb)
    return jnp.transpose(out128.reshape(L, B, TOK_DIM), (1, 0, 2))


# speedup vs baseline: 1.3555x; 1.3555x over previous
import functools

import jax
import jax.numpy as jnp
from jax import lax
from jax.experimental import pallas as pl
from jax.experimental.pallas import tpu as pltpu
from jax.experimental.pallas import tpu_sc as plsc

B = 4096
L = 200
N_TOK = B * L
EDIM = 16
TOK_DIM = 64
VOCAB = 1000000
GROWS = B * EDIM // 128

NUM_WORKERS = 32
PER_W = N_TOK // NUM_WORKERS
TBATCH = 128
K_INFLIGHT = 20
GROUP = TBATCH * K_INFLIGHT
NUM_GROUPS = PER_W // GROUP


@functools.cache
def _make_sc_gather():
    mesh = plsc.VectorSubcoreMesh(core_axis_name="c", subcore_axis_name="s")

    @functools.partial(
        pl.kernel,
        mesh=mesh,
        out_type=[
            jax.ShapeDtypeStruct((N_TOK, EDIM), jnp.float32),
            jax.ShapeDtypeStruct((N_TOK, EDIM), jnp.float32),
            jax.ShapeDtypeStruct((N_TOK // 8, EDIM), jnp.float32),
        ],
        scratch_types=[
            pltpu.VMEM((GROUP,), jnp.int32),
            pltpu.VMEM((GROUP,), jnp.int32),
            pltpu.VMEM((GROUP, EDIM), jnp.float32),
            pltpu.VMEM((GROUP, EDIM), jnp.float32),
            pltpu.VMEM((2 * GROUP,), jnp.float32),
            pltpu.VMEM((GROUP // 8, EDIM), jnp.float32),
            pltpu.SemaphoreType.DMA,
            pltpu.SemaphoreType.DMA,
        ],
        compiler_params=pltpu.CompilerParams(use_tc_tiling_on_sc=False),
    )
    def _sc_gather(ridx_hbm, aidx_hbm, c2_hbm, c3_hbm, rtab_hbm, atab_hbm,
                   out_r_hbm, out_a_hbm, ct_hbm, idxr_v, idxa_v, bufr_v,
                   bufa_v, cbuf_v, ctbuf_v, semr, sema):
        wid = lax.axis_index("s") * 2 + lax.axis_index("c")
        base = wid * PER_W
        lane = lax.iota(jnp.int32, 16)
        sel = lane & 1
        half = lane >> 1
        gdn = lax.GatherDimensionNumbers(
            offset_dims=(), collapsed_slice_dims=(0,), start_index_map=(0,))

        def _permute(vec, idx):
            return lax.gather(vec, idx[:, None], gdn, (1,),
                              mode=lax.GatherScatterMode.PROMISE_IN_BOUNDS)

        def group_body(g, carry):
            goff = base + g * GROUP
            pltpu.sync_copy(ridx_hbm.at[pl.ds(goff, GROUP)], idxr_v)
            pltpu.sync_copy(aidx_hbm.at[pl.ds(goff, GROUP)], idxa_v)
            rcopies = []
            acopies = []
            for t in range(K_INFLIGHT):
                sl = pl.ds(t * TBATCH, TBATCH)
                rcopies.append(
                    pltpu.async_copy(rtab_hbm.at[idxr_v.at[sl]], bufr_v.at[sl],
                                     semr))
                acopies.append(
                    pltpu.async_copy(atab_hbm.at[idxa_v.at[sl]], bufa_v.at[sl],
                                     sema))
            pltpu.sync_copy(c2_hbm.at[pl.ds(goff, GROUP)],
                            cbuf_v.at[pl.ds(0, GROUP)])
            pltpu.sync_copy(c3_hbm.at[pl.ds(goff, GROUP)],
                            cbuf_v.at[pl.ds(GROUP, GROUP)])

            def ct_row(r, crry):
                c2v = cbuf_v[pl.ds(8 * r, 16)]
                c3v = cbuf_v[pl.ds(GROUP + 8 * r, 16)]
                v0 = jnp.where(sel == 1, _permute(c3v, half),
                               _permute(c2v, half))
                v1 = jnp.where(sel == 1, _permute(c3v, half + 8),
                               _permute(c2v, half + 8))
                ctbuf_v[2 * r] = v0
                ctbuf_v[2 * r + 1] = v1
                return crry

            lax.fori_loop(0, GROUP // 16, ct_row, 0)
            pltpu.sync_copy(
                ctbuf_v, ct_hbm.at[pl.ds(base // 8 + g * (GROUP // 8),
                                         GROUP // 8)])
            for c in rcopies:
                c.wait()
            pltpu.sync_copy(bufr_v, out_r_hbm.at[pl.ds(goff, GROUP)])
            for c in acopies:
                c.wait()
            pltpu.sync_copy(bufa_v, out_a_hbm.at[pl.ds(goff, GROUP)])
            return carry

        lax.fori_loop(0, NUM_GROUPS, group_body, 0)

    return _sc_gather


def _proj_body(gr_ref, ga_ref, ct_ref, wr_ref, wa_ref, wc_ref, bb_ref,
               o_ref):
    acc = jnp.dot(gr_ref[...], wr_ref[...], preferred_element_type=jnp.float32)
    acc = acc + jnp.dot(ga_ref[...], wa_ref[...],
                        preferred_element_type=jnp.float32)
    acc = acc + jnp.dot(ct_ref[...], wc_ref[...],
                        preferred_element_type=jnp.float32)
    o_ref[...] = acc + bb_ref[...]


def _tc_project(gr128, ga128, ct512, wbd_r, wbd_a, wbd_c, bb):
    return pl.pallas_call(
        _proj_body,
        grid=(L,),
        in_specs=[
            pl.BlockSpec((GROWS, 128), lambda i: (i, 0)),
            pl.BlockSpec((GROWS, 128), lambda i: (i, 0)),
            pl.BlockSpec((GROWS, EDIM), lambda i: (i, 0)),
            pl.BlockSpec((128, 8 * TOK_DIM), lambda i: (0, 0)),
            pl.BlockSpec((128, 8 * TOK_DIM), lambda i: (0, 0)),
            pl.BlockSpec((EDIM, 8 * TOK_DIM), lambda i: (0, 0)),
            pl.BlockSpec((1, 8 * TOK_DIM), lambda i: (0, 0)),
        ],
        out_specs=pl.BlockSpec((GROWS, 8 * TOK_DIM), lambda i: (i, 0)),
        out_shape=jax.ShapeDtypeStruct((L * GROWS, 8 * TOK_DIM), jnp.float32),
        compiler_params=pltpu.CompilerParams(
            dimension_semantics=("arbitrary",),
        ),
    )(gr128, ga128, ct512, wbd_r, wbd_a, wbd_c, bb)


def kernel(x, resp_table, act_table, W, b):
    ridx = x[:, :, 0].T.reshape(N_TOK).astype(jnp.int32)
    aidx = x[:, :, 1].T.reshape(N_TOK).astype(jnp.int32)
    c2 = x[:, :, 2].T.reshape(N_TOK)
    c3 = x[:, :, 3].T.reshape(N_TOK)
    gr, ga, ct512 = _make_sc_gather()(ridx, aidx, c2, c3,
                                      resp_table, act_table)
    gr128 = gr.reshape(L * GROWS, 128)
    ga128 = ga.reshape(L * GROWS, 128)
    eye8 = jnp.eye(8, dtype=jnp.float32)
    wbd_r = jnp.kron(eye8, W[0:16])
    wbd_a = jnp.kron(eye8, W[16:32])
    wbd_c = jnp.kron(eye8, W[32:34])
    bb = jnp.tile(b, 8).reshape(1, 8 * TOK_DIM)
    out128 = _tc_project(gr128, ga128, ct512, wbd_r, wbd_a, wbd_c, bb)
    return jnp.transpose(out128.reshape(L, B, TOK_DIM), (1, 0, 2))
